# baseline (speedup 1.0000x reference)
import functools

import jax
import jax.numpy as jnp
from jax import lax
from jax.experimental import pallas as pl
from jax.experimental.pallas import tpu as pltpu
from jax.experimental.pallas import tpu_sc as plsc

V = 100000
EMB = 400
H = 1050
NL = 4
B = 32
T = 32
NTOK = B * T
G4 = 4 * H


EMBP = 512


def _sc_embed_gather(E_padded, idx_flat):
    mesh = plsc.VectorSubcoreMesh(core_axis_name="c", subcore_axis_name="s")
    NW = 32
    bpw = NTOK // NW

    @functools.partial(
        pl.kernel,
        out_type=jax.ShapeDtypeStruct((NTOK, EMBP), jnp.float32),
        mesh=mesh,
        scratch_types=[
            pltpu.VMEM((bpw,), jnp.int32),
            pltpu.VMEM((bpw, EMBP), jnp.float32),
            pltpu.SemaphoreType.DMA,
        ],
    )
    def gather_kernel(e_hbm, i_hbm, o_hbm, idx_v, rows_v, sem):
        wid = lax.axis_index("s") * 2 + lax.axis_index("c")
        base = wid * bpw
        pltpu.sync_copy(i_hbm.at[pl.ds(base, bpw)], idx_v)
        pltpu.async_copy(e_hbm.at[idx_v], rows_v, sem).wait()
        pltpu.sync_copy(rows_v, o_hbm.at[pl.ds(base, bpw)])

    return gather_kernel(E_padded, idx_flat)


_BG = 1024
_NG = (G4 + _BG - 1) // _BG
_G4P = _NG * _BG


def _layer_kernel(x_ref, wih_ref, whh_ref, b_ref, h0_ref, c0_ref,
                  ys_ref, hT_ref, cT_ref, gx_scr, h_scr, c_scr):
    j = pl.program_id(0)

    @pl.when(j < _NG)
    def _():
        din = wih_ref.shape[0]
        gx_scr[:, pl.ds(j * _BG, _BG)] = (
            jnp.dot(
                x_ref[:, 0:din], wih_ref[...], preferred_element_type=jnp.float32
            )
            + b_ref[...]
        )

    @pl.when(j == _NG)
    def _():
        h_scr[...] = h0_ref[...]
        c_scr[...] = c0_ref[...]

    @pl.when(j >= _NG)
    def _():
        t = j - _NG
        h = h_scr[...]
        g = gx_scr[pl.ds(t * B, B), 0:G4] + jnp.dot(
            h, whh_ref[...], preferred_element_type=jnp.float32
        )
        gi = jax.nn.sigmoid(g[:, 0:H])
        gf = jax.nn.sigmoid(g[:, H:2 * H])
        gg = jnp.tanh(g[:, 2 * H:3 * H])
        go = jax.nn.sigmoid(g[:, 3 * H:4 * H])
        c = gf * c_scr[...] + gi * gg
        h = go * jnp.tanh(c)
        h_scr[...] = h
        c_scr[...] = c
        ys_ref[...] = h

        @pl.when(t == T - 1)
        def _():
            hT_ref[...] = h
            cT_ref[...] = c


def _lstm_layer(xs, WihT, WhhT, bias2d, h0, c0):
    din_pad = xs.shape[1]
    din = WihT.shape[0]
    return pl.pallas_call(
        _layer_kernel,
        grid=(_NG + T,),
        in_specs=[
            pl.BlockSpec((NTOK, din_pad), lambda j: (0, 0)),
            pl.BlockSpec((din, _BG), lambda j: (0, jnp.minimum(j, _NG - 1))),
            pl.BlockSpec((H, G4), lambda j: (0, 0)),
            pl.BlockSpec((1, _BG), lambda j: (0, jnp.minimum(j, _NG - 1))),
            pl.BlockSpec((B, H), lambda j: (0, 0)),
            pl.BlockSpec((B, H), lambda j: (0, 0)),
        ],
        out_specs=[
            pl.BlockSpec((B, H), lambda j: (jnp.maximum(j - _NG, 0), 0)),
            pl.BlockSpec((B, H), lambda j: (0, 0)),
            pl.BlockSpec((B, H), lambda j: (0, 0)),
        ],
        out_shape=[
            jax.ShapeDtypeStruct((NTOK, H), jnp.float32),
            jax.ShapeDtypeStruct((B, H), jnp.float32),
            jax.ShapeDtypeStruct((B, H), jnp.float32),
        ],
        scratch_shapes=[
            pltpu.VMEM((NTOK, _G4P), jnp.float32),
            pltpu.VMEM((B, H), jnp.float32),
            pltpu.VMEM((B, H), jnp.float32),
        ],
        compiler_params=pltpu.CompilerParams(
            dimension_semantics=("arbitrary",),
        ),
    )(xs, WihT, WhhT, bias2d, h0, c0)


_BV = 2048
_NV = (V + _BV - 1) // _BV


def _proj_kernel(ys_ref, w_ref, b_ref, o_ref, x_scr):
    v = pl.program_id(0)

    @pl.when(v == 0)
    def _():
        ys = ys_ref[...].reshape(T, B, H)
        x_scr[...] = jnp.swapaxes(ys, 0, 1).reshape(NTOK, H)

    p = jnp.dot(x_scr[...], w_ref[...], preferred_element_type=jnp.float32)
    o_ref[...] = p.T + b_ref[...]


def _vocab_proj(ys, WoutT, b_col):
    return pl.pallas_call(
        _proj_kernel,
        grid=(_NV,),
        in_specs=[
            pl.BlockSpec((NTOK, H), lambda v: (0, 0)),
            pl.BlockSpec((H, _BV), lambda v: (0, v)),
            pl.BlockSpec((_BV, 1), lambda v: (v, 0)),
        ],
        out_specs=pl.BlockSpec((_BV, NTOK), lambda v: (v, 0)),
        out_shape=jax.ShapeDtypeStruct((V, NTOK), jnp.float32),
        scratch_shapes=[pltpu.VMEM((NTOK, H), jnp.float32)],
        compiler_params=pltpu.CompilerParams(
            dimension_semantics=("arbitrary",),
        ),
    )(ys, WoutT, b_col)


def kernel(x, h0, c0, E, W_ih0, W_hh0, b_ih0, b_hh0, W_ih1, W_hh1, b_ih1, b_hh1, W_ih2, W_hh2, b_ih2, b_hh2, W_ih3, W_hh3, b_ih3, b_hh3, W_out, b_out):
    idx = x.astype(jnp.int32).T.reshape(NTOK)
    E_padded = jnp.pad(E, ((0, 0), (0, EMBP - EMB)))
    xs = _sc_embed_gather(E_padded, idx)

    hs = []
    cs = []
    for l, (W_ih, W_hh, b_ih, b_hh) in enumerate((
        (W_ih0, W_hh0, b_ih0, b_hh0),
        (W_ih1, W_hh1, b_ih1, b_hh1),
        (W_ih2, W_hh2, b_ih2, b_hh2),
        (W_ih3, W_hh3, b_ih3, b_hh3),
    )):
        xs, hT, cT = _lstm_layer(
            xs, W_ih.T, W_hh.T, (b_ih + b_hh).reshape(1, G4), h0[l], c0[l]
        )
        hs.append(hT)
        cs.append(cT)

    logits_t = _vocab_proj(xs, W_out.T, b_out.reshape(V, 1))
    return logits_t.T, (jnp.stack(hs), jnp.stack(cs))


# speedup vs baseline: 1.8354x; 1.8354x over previous
import functools

import jax
import jax.numpy as jnp
from jax import lax
from jax.experimental import pallas as pl
from jax.experimental.pallas import tpu as pltpu
from jax.experimental.pallas import tpu_sc as plsc

V = 100000
EMB = 400
H = 1050
NL = 4
B = 32
T = 32
NTOK = B * T
G4 = 4 * H


def _sc_embed_gather(E_rowmajor, idx_flat):
    mesh = plsc.ScalarSubcoreMesh(axis_name="core", num_cores=2)
    npc = NTOK // 2
    pieces = ((0, 128), (128, 128), (256, 128), (384, 16))

    @functools.partial(
        pl.kernel,
        out_type=jax.ShapeDtypeStruct((NTOK, EMB), jnp.float32),
        mesh=mesh,
        scratch_types=[
            pltpu.SMEM((npc,), jnp.int32),
            pltpu.SemaphoreType.DMA,
        ],
    )
    def gather_kernel(e_hbm, i_hbm, o_hbm, idx_s, sem):
        cid = lax.axis_index("core")
        base = cid * npc
        pltpu.async_copy(i_hbm.at[pl.ds(base, npc)], idx_s, sem).wait()

        @pl.loop(0, npc)
        def _(j):
            tok = idx_s[j]
            row = base + j
            for off, w in pieces:
                pltpu.async_copy(
                    e_hbm.at[tok, pl.ds(off, w)],
                    o_hbm.at[row, pl.ds(off, w)],
                    sem,
                )

        @pl.loop(0, npc)
        def _(j):
            row = base + j
            for off, w in pieces:
                pltpu.make_async_copy(
                    e_hbm.at[0, pl.ds(off, w)],
                    o_hbm.at[row, pl.ds(off, w)],
                    sem,
                ).wait()

    return gather_kernel(E_rowmajor, idx_flat)


_BG = 1024
_NG = (G4 + _BG - 1) // _BG
_G4P = _NG * _BG


def _layer_kernel(x_ref, wih_ref, whh_ref, b_ref, h0_ref, c0_ref,
                  ys_ref, hT_ref, cT_ref, gx_scr, h_scr, c_scr):
    j = pl.program_id(0)

    @pl.when(j < _NG)
    def _():
        din = wih_ref.shape[0]
        gx_scr[:, pl.ds(j * _BG, _BG)] = (
            jnp.dot(
                x_ref[:, 0:din], wih_ref[...], preferred_element_type=jnp.float32
            )
            + b_ref[...]
        )

    @pl.when(j == _NG)
    def _():
        h_scr[...] = h0_ref[...]
        c_scr[...] = c0_ref[...]

    @pl.when(j >= _NG)
    def _():
        t = j - _NG
        h = h_scr[...]
        g = gx_scr[pl.ds(t * B, B), 0:G4] + jnp.dot(
            h, whh_ref[...], preferred_element_type=jnp.float32
        )
        gi = jax.nn.sigmoid(g[:, 0:H])
        gf = jax.nn.sigmoid(g[:, H:2 * H])
        gg = jnp.tanh(g[:, 2 * H:3 * H])
        go = jax.nn.sigmoid(g[:, 3 * H:4 * H])
        c = gf * c_scr[...] + gi * gg
        h = go * jnp.tanh(c)
        h_scr[...] = h
        c_scr[...] = c
        ys_ref[...] = h

        @pl.when(t == T - 1)
        def _():
            hT_ref[...] = h
            cT_ref[...] = c


def _lstm_layer(xs, WihT, WhhT, bias2d, h0, c0):
    din_pad = xs.shape[1]
    din = WihT.shape[0]
    return pl.pallas_call(
        _layer_kernel,
        grid=(_NG + T,),
        in_specs=[
            pl.BlockSpec((NTOK, din_pad), lambda j: (0, 0)),
            pl.BlockSpec((din, _BG), lambda j: (0, jnp.minimum(j, _NG - 1))),
            pl.BlockSpec((H, G4), lambda j: (0, 0)),
            pl.BlockSpec((1, _BG), lambda j: (0, jnp.minimum(j, _NG - 1))),
            pl.BlockSpec((B, H), lambda j: (0, 0)),
            pl.BlockSpec((B, H), lambda j: (0, 0)),
        ],
        out_specs=[
            pl.BlockSpec((B, H), lambda j: (jnp.maximum(j - _NG, 0), 0)),
            pl.BlockSpec((B, H), lambda j: (0, 0)),
            pl.BlockSpec((B, H), lambda j: (0, 0)),
        ],
        out_shape=[
            jax.ShapeDtypeStruct((NTOK, H), jnp.float32),
            jax.ShapeDtypeStruct((B, H), jnp.float32),
            jax.ShapeDtypeStruct((B, H), jnp.float32),
        ],
        scratch_shapes=[
            pltpu.VMEM((NTOK, _G4P), jnp.float32),
            pltpu.VMEM((B, H), jnp.float32),
            pltpu.VMEM((B, H), jnp.float32),
        ],
        compiler_params=pltpu.CompilerParams(
            dimension_semantics=("arbitrary",),
        ),
    )(xs, WihT, WhhT, bias2d, h0, c0)


_BV = 2048
_NV = (V + _BV - 1) // _BV


def _proj_kernel(ys_ref, w_ref, b_ref, o_ref, x_scr):
    v = pl.program_id(0)

    @pl.when(v == 0)
    def _():
        ys = ys_ref[...].reshape(T, B, H)
        x_scr[...] = jnp.swapaxes(ys, 0, 1).reshape(NTOK, H)

    p = jnp.dot(x_scr[...], w_ref[...], preferred_element_type=jnp.float32)
    o_ref[...] = p.T + b_ref[...]


def _vocab_proj(ys, WoutT, b_col):
    return pl.pallas_call(
        _proj_kernel,
        grid=(_NV,),
        in_specs=[
            pl.BlockSpec((NTOK, H), lambda v: (0, 0)),
            pl.BlockSpec((H, _BV), lambda v: (0, v)),
            pl.BlockSpec((_BV, 1), lambda v: (v, 0)),
        ],
        out_specs=pl.BlockSpec((_BV, NTOK), lambda v: (v, 0)),
        out_shape=jax.ShapeDtypeStruct((V, NTOK), jnp.float32),
        scratch_shapes=[pltpu.VMEM((NTOK, H), jnp.float32)],
        compiler_params=pltpu.CompilerParams(
            dimension_semantics=("arbitrary",),
        ),
    )(ys, WoutT, b_col)


def kernel(x, h0, c0, E, W_ih0, W_hh0, b_ih0, b_hh0, W_ih1, W_hh1, b_ih1, b_hh1, W_ih2, W_hh2, b_ih2, b_hh2, W_ih3, W_hh3, b_ih3, b_hh3, W_out, b_out):
    idx = x.astype(jnp.int32).T.reshape(NTOK)
    xs = _sc_embed_gather(E, idx)

    hs = []
    cs = []
    for l, (W_ih, W_hh, b_ih, b_hh) in enumerate((
        (W_ih0, W_hh0, b_ih0, b_hh0),
        (W_ih1, W_hh1, b_ih1, b_hh1),
        (W_ih2, W_hh2, b_ih2, b_hh2),
        (W_ih3, W_hh3, b_ih3, b_hh3),
    )):
        xs, hT, cT = _lstm_layer(
            xs, W_ih.T, W_hh.T, (b_ih + b_hh).reshape(1, G4), h0[l], c0[l]
        )
        hs.append(hT)
        cs.append(cT)

    logits_t = _vocab_proj(xs, W_out.T, b_out.reshape(V, 1))
    return logits_t.T, (jnp.stack(hs), jnp.stack(cs))


# speedup vs baseline: 1.8410x; 1.0030x over previous
import functools

import jax
import jax.numpy as jnp
from jax import lax
from jax.experimental import pallas as pl
from jax.experimental.pallas import tpu as pltpu
from jax.experimental.pallas import tpu_sc as plsc

V = 100000
EMB = 400
H = 1050
NL = 4
B = 32
T = 32
NTOK = B * T
G4 = 4 * H


def _sc_embed_gather(E_rowmajor, idx_flat):
    mesh = plsc.ScalarSubcoreMesh(axis_name="core", num_cores=2)
    npc = NTOK // 2
    pieces = ((0, 128), (128, 128), (256, 128), (384, 16))

    @functools.partial(
        pl.kernel,
        out_type=jax.ShapeDtypeStruct((NTOK, EMB), jnp.float32),
        mesh=mesh,
        scratch_types=[
            pltpu.SMEM((npc,), jnp.int32),
            pltpu.SemaphoreType.DMA,
        ],
    )
    def gather_kernel(e_hbm, i_hbm, o_hbm, idx_s, sem):
        cid = lax.axis_index("core")
        base = cid * npc
        pltpu.async_copy(i_hbm.at[pl.ds(base, npc)], idx_s, sem).wait()

        @pl.loop(0, npc)
        def _(j):
            tok = idx_s[j]
            row = base + j
            for off, w in pieces:
                pltpu.async_copy(
                    e_hbm.at[tok, pl.ds(off, w)],
                    o_hbm.at[row, pl.ds(off, w)],
                    sem,
                )

        @pl.loop(0, npc)
        def _(j):
            row = base + j
            for off, w in pieces:
                pltpu.make_async_copy(
                    e_hbm.at[0, pl.ds(off, w)],
                    o_hbm.at[row, pl.ds(off, w)],
                    sem,
                ).wait()

    return gather_kernel(E_rowmajor, idx_flat)


_BG = 1024
_NG = (G4 + _BG - 1) // _BG
_G4P = _NG * _BG


def _layer_kernel(x_ref, wih_ref, whh_ref, b_ref, h0_ref, c0_ref,
                  ys_ref, hT_ref, cT_ref, gx_scr, h_scr, c_scr):
    j = pl.program_id(0)

    @pl.when(j < _NG)
    def _():
        din = wih_ref.shape[0]
        gx_scr[:, pl.ds(j * _BG, _BG)] = (
            jnp.dot(
                x_ref[:, 0:din], wih_ref[...], preferred_element_type=jnp.float32
            )
            + b_ref[...]
        )

    @pl.when(j == _NG)
    def _():
        h_scr[...] = h0_ref[...]
        c_scr[...] = c0_ref[...]

    @pl.when(j >= _NG)
    def _():
        t = j - _NG
        h = h_scr[...]
        g = gx_scr[pl.ds(t * B, B), 0:G4] + jnp.dot(
            h, whh_ref[...], preferred_element_type=jnp.float32
        )
        gi = jax.nn.sigmoid(g[:, 0:H])
        gf = jax.nn.sigmoid(g[:, H:2 * H])
        gg = jnp.tanh(g[:, 2 * H:3 * H])
        go = jax.nn.sigmoid(g[:, 3 * H:4 * H])
        c = gf * c_scr[...] + gi * gg
        h = go * jnp.tanh(c)
        h_scr[...] = h
        c_scr[...] = c
        ys_ref[...] = h

        @pl.when(t == T - 1)
        def _():
            hT_ref[...] = h
            cT_ref[...] = c


def _lstm_layer(xs, WihT, WhhT, bias2d, h0, c0):
    din_pad = xs.shape[1]
    din = WihT.shape[0]
    return pl.pallas_call(
        _layer_kernel,
        grid=(_NG + T,),
        in_specs=[
            pl.BlockSpec((NTOK, din_pad), lambda j: (0, 0)),
            pl.BlockSpec((din, _BG), lambda j: (0, jnp.minimum(j, _NG - 1))),
            pl.BlockSpec((H, G4), lambda j: (0, 0)),
            pl.BlockSpec((1, _BG), lambda j: (0, jnp.minimum(j, _NG - 1))),
            pl.BlockSpec((B, H), lambda j: (0, 0)),
            pl.BlockSpec((B, H), lambda j: (0, 0)),
        ],
        out_specs=[
            pl.BlockSpec((B, H), lambda j: (jnp.maximum(j - _NG, 0), 0)),
            pl.BlockSpec((B, H), lambda j: (0, 0)),
            pl.BlockSpec((B, H), lambda j: (0, 0)),
        ],
        out_shape=[
            jax.ShapeDtypeStruct((NTOK, H), jnp.float32),
            jax.ShapeDtypeStruct((B, H), jnp.float32),
            jax.ShapeDtypeStruct((B, H), jnp.float32),
        ],
        scratch_shapes=[
            pltpu.VMEM((NTOK, _G4P), jnp.float32),
            pltpu.VMEM((B, H), jnp.float32),
            pltpu.VMEM((B, H), jnp.float32),
        ],
        compiler_params=pltpu.CompilerParams(
            dimension_semantics=("arbitrary",),
        ),
    )(xs, WihT, WhhT, bias2d, h0, c0)


_BV = 2560
_NV = (V + _BV - 1) // _BV


def _proj_kernel(ys_ref, w_ref, b_ref, o_ref, x_scr):
    v = pl.program_id(0)

    @pl.when(v == 0)
    def _():
        ys = ys_ref[...].reshape(T, B, H)
        x_scr[...] = jnp.swapaxes(ys, 0, 1).reshape(NTOK, H)

    p = jnp.dot(x_scr[...], w_ref[...], preferred_element_type=jnp.float32)
    o_ref[...] = p.T + b_ref[...]


def _vocab_proj(ys, WoutT, b_col):
    return pl.pallas_call(
        _proj_kernel,
        grid=(_NV,),
        in_specs=[
            pl.BlockSpec((NTOK, H), lambda v: (0, 0)),
            pl.BlockSpec((H, _BV), lambda v: (0, v)),
            pl.BlockSpec((_BV, 1), lambda v: (v, 0)),
        ],
        out_specs=pl.BlockSpec((_BV, NTOK), lambda v: (v, 0)),
        out_shape=jax.ShapeDtypeStruct((V, NTOK), jnp.float32),
        scratch_shapes=[pltpu.VMEM((NTOK, H), jnp.float32)],
        compiler_params=pltpu.CompilerParams(
            dimension_semantics=("arbitrary",),
        ),
    )(ys, WoutT, b_col)


def kernel(x, h0, c0, E, W_ih0, W_hh0, b_ih0, b_hh0, W_ih1, W_hh1, b_ih1, b_hh1, W_ih2, W_hh2, b_ih2, b_hh2, W_ih3, W_hh3, b_ih3, b_hh3, W_out, b_out):
    idx = x.astype(jnp.int32).T.reshape(NTOK)
    xs = _sc_embed_gather(E, idx)

    hs = []
    cs = []
    for l, (W_ih, W_hh, b_ih, b_hh) in enumerate((
        (W_ih0, W_hh0, b_ih0, b_hh0),
        (W_ih1, W_hh1, b_ih1, b_hh1),
        (W_ih2, W_hh2, b_ih2, b_hh2),
        (W_ih3, W_hh3, b_ih3, b_hh3),
    )):
        xs, hT, cT = _lstm_layer(
            xs, W_ih.T, W_hh.T, (b_ih + b_hh).reshape(1, G4), h0[l], c0[l]
        )
        hs.append(hT)
        cs.append(cT)

    logits_t = _vocab_proj(xs, W_out.T, b_out.reshape(V, 1))
    return logits_t.T, (jnp.stack(hs), jnp.stack(cs))
